# baseline (speedup 1.0000x reference)
import functools

import jax
import jax.numpy as jnp
from jax import lax
from jax.experimental import pallas as pl
from jax.experimental.pallas import tpu as pltpu
from jax.experimental.pallas import tpu_sc as plsc

VOCAB = 1000
VPAD = 1024
EMB = 128
B = 4096
L = 20
N = B * L

NC = 2
NS = 16
NW = NC * NS
ROWS_W = N // NW
CHUNK = 64
LANES = 16


def _table_body(emb_ref, wt_ref, b_ref, out_ref):
    g = jnp.dot(emb_ref[...], wt_ref[...], preferred_element_type=jnp.float32)
    g = g + b_ref[...]
    m = jnp.max(g, axis=1, keepdims=True)
    e = jnp.exp(g - m)
    lse = jnp.log(jnp.sum(e, axis=1, keepdims=True))
    out_ref[...] = g - (m + lse)


def _compute_table(embedding, fc1_w, fc1_b):
    wt_pad = jnp.pad(fc1_w.T, ((0, 0), (0, VPAD - VOCAB)))
    b_pad = jnp.pad(
        fc1_b.reshape(1, VOCAB), ((0, 0), (0, VPAD - VOCAB)),
        constant_values=-1e30,
    )
    return pl.pallas_call(
        _table_body,
        out_shape=jax.ShapeDtypeStruct((VOCAB, VPAD), jnp.float32),
    )(embedding, wt_pad, b_pad)


def _gather_body(x_hbm, m_hbm, tab_hbm, out_hbm, xv, mv, rows_v, sem):
    wid = lax.axis_index("s") * NC + lax.axis_index("c")
    base = wid * ROWS_W
    pltpu.sync_copy(x_hbm.at[pl.ds(base, ROWS_W)], xv)
    pltpu.sync_copy(m_hbm.at[pl.ds(base, ROWS_W)], mv)

    def mul_body(i, carry):
        s = pl.ds(i * LANES, LANES)
        xv[s] = xv[s] * mv[s]
        return carry

    lax.fori_loop(0, ROWS_W // LANES, mul_body, 0, unroll=4)

    def chunk_body(j, carry):
        idx = xv.at[pl.ds(j * CHUNK, CHUNK)]
        pltpu.async_copy(tab_hbm.at[idx], rows_v, sem).wait()
        pltpu.sync_copy(rows_v, out_hbm.at[pl.ds(base + j * CHUNK, CHUNK)])
        return carry

    lax.fori_loop(0, ROWS_W // CHUNK, chunk_body, 0)


_sc_gather = functools.partial(
    pl.kernel,
    out_type=jax.ShapeDtypeStruct((N, VPAD), jnp.float32),
    mesh=plsc.VectorSubcoreMesh(
        core_axis_name="c", subcore_axis_name="s", num_cores=NC, num_subcores=NS
    ),
    scratch_types=[
        pltpu.VMEM((ROWS_W,), jnp.int32),
        pltpu.VMEM((ROWS_W,), jnp.int32),
        pltpu.VMEM((CHUNK, VPAD), jnp.float32),
        pltpu.SemaphoreType.DMA,
    ],
)(_gather_body)


def kernel(x, mask, embedding, fc1_w, fc1_b):
    table = _compute_table(embedding, fc1_w, fc1_b)
    x_flat = x.reshape(N).astype(jnp.int32)
    m_flat = mask.reshape(N).astype(jnp.int32)
    out = _sc_gather(x_flat, m_flat, table)
    return out.reshape(B, L, VPAD)[:, :, :VOCAB]
